# Initial kernel scaffold; baseline (speedup 1.0000x reference)
import jax
import jax.numpy as jnp
from jax.experimental import pallas as pl


def kernel(x, edge_index, W1, a_src1, a_dst1, b1, W2, a_src2, a_dst2, b2, Wh, bh):
    raise NotImplementedError("write your pallas kernel here")


# speedup vs baseline: 15.3201x; 15.3201x over previous
import functools

import jax
import jax.numpy as jnp
from jax import lax
from jax.experimental import pallas as pl
from jax.experimental.pallas import tpu as pltpu
from jax.experimental.pallas import tpu_sc as plsc

_NC = 2
_NS = 16
_NW = _NC * _NS
_L = 16


def _sc_edge_pass(h, alpha_s, alpha_d, src, dst):
  n, d_dim = h.shape
  e_num = src.shape[0]
  epw = e_num // _NW
  k = 80
  n_chunks = epw // k
  assert epw % k == 0 and k % _L == 0
  stripe_rows = (n // _NS) // 8 * 8
  tail_rows = n - stripe_rows * _NS
  assert tail_rows % 8 == 0 and tail_rows <= k

  mesh = plsc.VectorSubcoreMesh(core_axis_name="c", subcore_axis_name="s")

  @functools.partial(
      pl.kernel,
      out_type=(
          jax.ShapeDtypeStruct((_NC, n, d_dim), jnp.float32),
          jax.ShapeDtypeStruct((_NC, n, _L), jnp.float32),
      ),
      mesh=mesh,
      compiler_params=pltpu.CompilerParams(
          needs_layout_passes=False, use_tc_tiling_on_sc=False),
      scratch_types=[
          pltpu.VMEM((n,), jnp.float32),
          pltpu.VMEM((n,), jnp.float32),
          pltpu.VMEM((k,), jnp.int32),
          pltpu.VMEM((k,), jnp.int32),
          pltpu.VMEM((k, d_dim), jnp.float32),
          pltpu.VMEM((k, _L), jnp.float32),
          pltpu.VMEM_SHARED((n, d_dim), jnp.float32),
          pltpu.VMEM_SHARED((n, _L), jnp.float32),
          pltpu.SemaphoreType.DMA,
      ],
  )
  def edge_kernel(h_hbm, as_hbm, ad_hbm, src_hbm, dst_hbm,
                  agg_out, den_out,
                  as_v, ad_v, src_v, dst_v, rows_v, dmsg_v,
                  agg_sh, den_sh, sem):
    cid = lax.axis_index("c")
    sid = lax.axis_index("s")
    wid = sid * _NC + cid
    pltpu.sync_copy(as_hbm, as_v)
    pltpu.sync_copy(ad_hbm, ad_v)

    zeros = jnp.zeros((_L,), jnp.float32)

    def zero_row(i, _):
      for j in range(d_dim // _L):
        rows_v[i, pl.ds(j * _L, _L)] = zeros
      dmsg_v[i, :] = zeros
      return 0

    lax.fori_loop(0, k, zero_row, 0)

    stripe = sid * stripe_rows
    nfull = stripe_rows // k
    rem = stripe_rows - nfull * k
    for q in range(nfull):
      pltpu.sync_copy(rows_v, agg_sh.at[pl.ds(stripe + q * k, k)])
      pltpu.sync_copy(dmsg_v, den_sh.at[pl.ds(stripe + q * k, k)])
    if rem:
      pltpu.sync_copy(rows_v.at[pl.ds(0, rem)],
                      agg_sh.at[pl.ds(stripe + nfull * k, rem)])
      pltpu.sync_copy(dmsg_v.at[pl.ds(0, rem)],
                      den_sh.at[pl.ds(stripe + nfull * k, rem)])

    @pl.when(sid == 0)
    def _zero_tail():
      tail = stripe_rows * _NS
      pltpu.sync_copy(rows_v.at[pl.ds(0, tail_rows)],
                      agg_sh.at[pl.ds(tail, tail_rows)])
      pltpu.sync_copy(dmsg_v.at[pl.ds(0, tail_rows)],
                      den_sh.at[pl.ds(tail, tail_rows)])

    plsc.subcore_barrier()

    def chunk_body(c, _):
      base = wid * epw + c * k
      pltpu.sync_copy(src_hbm.at[pl.ds(base, k)], src_v)
      pltpu.sync_copy(dst_hbm.at[pl.ds(base, k)], dst_v)
      pltpu.async_copy(h_hbm.at[src_v], rows_v, sem).wait()

      def row_body(r, _):
        rsp = jnp.full((_L,), r, jnp.int32)
        sidx = plsc.load_gather(src_v, [rsp])
        didx = plsc.load_gather(dst_v, [rsp])
        logit = plsc.load_gather(as_v, [sidx]) + plsc.load_gather(ad_v, [didx])
        w = jnp.exp(jnp.maximum(logit, 0.2 * logit))
        for j in range(d_dim // _L):
          rows_v[r, pl.ds(j * _L, _L)] = rows_v[r, pl.ds(j * _L, _L)] * w
        dmsg_v[r, :] = w
        return 0

      lax.fori_loop(0, k, row_body, 0)

      pltpu.sync_copy(rows_v, agg_sh.at[dst_v], add=True)
      pltpu.sync_copy(dmsg_v, den_sh.at[dst_v], add=True)
      return 0

    lax.fori_loop(0, n_chunks, chunk_body, 0)

    plsc.subcore_barrier()
    pltpu.sync_copy(agg_sh.at[pl.ds(stripe, stripe_rows)],
                    agg_out.at[cid, pl.ds(stripe, stripe_rows)])
    pltpu.sync_copy(den_sh.at[pl.ds(stripe, stripe_rows)],
                    den_out.at[cid, pl.ds(stripe, stripe_rows)])

    @pl.when(sid == 0)
    def _copy_tail():
      tail = stripe_rows * _NS
      pltpu.sync_copy(agg_sh.at[pl.ds(tail, tail_rows)],
                      agg_out.at[cid, pl.ds(tail, tail_rows)])
      pltpu.sync_copy(den_sh.at[pl.ds(tail, tail_rows)],
                      den_out.at[cid, pl.ds(tail, tail_rows)])

  return edge_kernel(h, alpha_s, alpha_d, src, dst)


def _prep_body(x_ref, w_ref, asrc_ref, adst_ref, h_ref, s_ref, d_ref):
  h = jnp.dot(x_ref[...], w_ref[...], preferred_element_type=jnp.float32)
  h_ref[...] = h
  s_ref[...] = jnp.sum(h * asrc_ref[...], axis=1, keepdims=True)
  d_ref[...] = jnp.sum(h * adst_ref[...], axis=1, keepdims=True)


def _prep(x, w, a_src, a_dst):
  n, d_dim = x.shape
  return pl.pallas_call(
      _prep_body,
      out_shape=(
          jax.ShapeDtypeStruct((n, d_dim), jnp.float32),
          jax.ShapeDtypeStruct((n, 1), jnp.float32),
          jax.ShapeDtypeStruct((n, 1), jnp.float32),
      ),
  )(x, w, a_src.reshape(1, d_dim), a_dst.reshape(1, d_dim))


def _normalize(aggp_ref, denp_ref, h_ref, s_ref, d_ref, b_ref):
  logit = s_ref[...] + d_ref[...]
  w_self = jnp.exp(jnp.maximum(logit, 0.2 * logit))
  agg = aggp_ref[0] + aggp_ref[1] + w_self * h_ref[...]
  den = denp_ref[0, :, 0:1] + denp_ref[1, :, 0:1] + w_self
  return jnp.maximum(agg / (den + 1e-16) + b_ref[...], 0.0)


def _combine_body(aggp_ref, denp_ref, h_ref, s_ref, d_ref, b_ref,
                  w2_ref, asrc_ref, adst_ref, h2_ref, s2_ref, d2_ref):
  out1 = _normalize(aggp_ref, denp_ref, h_ref, s_ref, d_ref, b_ref)
  h2 = jnp.dot(out1, w2_ref[...], preferred_element_type=jnp.float32)
  h2_ref[...] = h2
  s2_ref[...] = jnp.sum(h2 * asrc_ref[...], axis=1, keepdims=True)
  d2_ref[...] = jnp.sum(h2 * adst_ref[...], axis=1, keepdims=True)


def _combine(aggp, denp, h, s, d, b, w2, a_src2, a_dst2):
  n, d_dim = h.shape
  return pl.pallas_call(
      _combine_body,
      out_shape=(
          jax.ShapeDtypeStruct((n, d_dim), jnp.float32),
          jax.ShapeDtypeStruct((n, 1), jnp.float32),
          jax.ShapeDtypeStruct((n, 1), jnp.float32),
      ),
  )(aggp, denp, h, s, d, b.reshape(1, d_dim),
    w2, a_src2.reshape(1, d_dim), a_dst2.reshape(1, d_dim))


def _final_body(aggp_ref, denp_ref, h_ref, s_ref, d_ref, b_ref,
                wh_ref, bh_ref, out_ref):
  out2 = _normalize(aggp_ref, denp_ref, h_ref, s_ref, d_ref, b_ref)
  out_ref[...] = (
      jnp.dot(out2, wh_ref[...], preferred_element_type=jnp.float32)
      + bh_ref[...])


def _final(aggp, denp, h, s, d, b, wh, bh):
  n, d_dim = h.shape
  d_out = wh.shape[1]
  return pl.pallas_call(
      _final_body,
      out_shape=jax.ShapeDtypeStruct((n, d_out), jnp.float32),
  )(aggp, denp, h, s, d, b.reshape(1, d_dim), wh, bh.reshape(1, d_out))


def kernel(x, edge_index, W1, a_src1, a_dst1, b1,
           W2, a_src2, a_dst2, b2, Wh, bh):
  n = x.shape[0]
  src = edge_index[0]
  dst = edge_index[1]

  h1, s1, d1 = _prep(x, W1, a_src1, a_dst1)
  aggp1, denp1 = _sc_edge_pass(h1, s1.reshape(n), d1.reshape(n), src, dst)
  h2, s2, d2 = _combine(aggp1, denp1, h1, s1, d1, b1, W2, a_src2, a_dst2)
  aggp2, denp2 = _sc_edge_pass(h2, s2.reshape(n), d2.reshape(n), src, dst)
  return _final(aggp2, denp2, h2, s2, d2, b2, Wh, bh)
